# baseline (speedup 1.0000x reference)
import functools

import jax
import jax.numpy as jnp
from jax import lax
from jax.experimental import pallas as pl
from jax.experimental.pallas import tpu as pltpu
from jax.experimental.pallas import tpu_sc as plsc

BATCH = 16384
EMB = 16
ROWS_PER_GROUP = 128 // EMB
NUM_CORES = 2
NUM_SUBCORES = 16
NUM_WORKERS = NUM_CORES * NUM_SUBCORES
ROWS_PER_WORKER = BATCH // NUM_WORKERS
CHUNK = 128
NUM_CHUNKS = ROWS_PER_WORKER // CHUNK


def kernel(user_idx, item_idx, user_emb, item_emb):
    num_users, num_items = user_emb.shape[0], item_emb.shape[0]
    uemb128 = user_emb.reshape(num_users // ROWS_PER_GROUP, 128)
    iemb128 = item_emb.reshape(num_items // ROWS_PER_GROUP, 128)
    ug = (user_idx // ROWS_PER_GROUP).astype(jnp.int32)
    ig = (item_idx // ROWS_PER_GROUP).astype(jnp.int32)
    uro = ((user_idx % ROWS_PER_GROUP) * EMB).astype(jnp.int32)
    iro = ((item_idx % ROWS_PER_GROUP) * EMB).astype(jnp.int32)

    mesh = plsc.VectorSubcoreMesh(core_axis_name="c", subcore_axis_name="s")

    @functools.partial(
        pl.kernel,
        out_type=jax.ShapeDtypeStruct((BATCH // ROWS_PER_GROUP, 128), jnp.float32),
        mesh=mesh,
        scratch_types=[
            pltpu.VMEM((NUM_CHUNKS, CHUNK), jnp.int32),
            pltpu.VMEM((NUM_CHUNKS, CHUNK), jnp.int32),
            pltpu.VMEM((NUM_CHUNKS, CHUNK), jnp.int32),
            pltpu.VMEM((NUM_CHUNKS, CHUNK), jnp.int32),
            pltpu.VMEM((2, CHUNK, 128), jnp.float32),
            pltpu.VMEM((2, CHUNK, 128), jnp.float32),
            pltpu.VMEM((ROWS_PER_WORKER // ROWS_PER_GROUP, 128), jnp.float32),
            pltpu.SemaphoreType.DMA,
            pltpu.SemaphoreType.DMA,
        ],
    )
    def gmf(ug_hbm, ig_hbm, uro_hbm, iro_hbm, uemb_hbm, iemb_hbm, out_hbm,
            ug_v, ig_v, uro_v, iro_v, gu_v, gi_v, out_v, sem_u, sem_i):
        wid = lax.axis_index("s") * NUM_CORES + lax.axis_index("c")
        base = wid * ROWS_PER_WORKER

        for c in range(NUM_CHUNKS):
            off = base + c * CHUNK
            pltpu.sync_copy(ug_hbm.at[pl.ds(off, CHUNK)], ug_v.at[c])
            pltpu.sync_copy(ig_hbm.at[pl.ds(off, CHUNK)], ig_v.at[c])
            pltpu.sync_copy(uro_hbm.at[pl.ds(off, CHUNK)], uro_v.at[c])
            pltpu.sync_copy(iro_hbm.at[pl.ds(off, CHUNK)], iro_v.at[c])

        def fire(c):
            b = c % 2
            cu = pltpu.async_copy(uemb_hbm.at[ug_v.at[c]], gu_v.at[b], sem_u)
            ci = pltpu.async_copy(iemb_hbm.at[ig_v.at[c]], gi_v.at[b], sem_i)
            return cu, ci

        pending = fire(0)
        for c in range(NUM_CHUNKS):
            cu, ci = pending
            if c + 1 < NUM_CHUNKS:
                nxt = fire(c + 1)
            cu.wait()
            ci.wait()
            b = c % 2

            @pl.loop(0, CHUNK, step=16)
            def _(r0):
                uofs16 = uro_v[c, pl.ds(r0, 16)]
                iofs16 = iro_v[c, pl.ds(r0, 16)]
                for j in range(16):
                    prow = c * (CHUNK // 8) + r0 // 8 + j // 8
                    out_v[prow, pl.ds((j % 8) * EMB, EMB)] = (
                        gu_v[b, r0 + j, pl.ds(uofs16[j], EMB)]
                        * gi_v[b, r0 + j, pl.ds(iofs16[j], EMB)]
                    )

            if c + 1 < NUM_CHUNKS:
                pending = nxt

        pltpu.sync_copy(
            out_v,
            out_hbm.at[pl.ds(wid * (ROWS_PER_WORKER // ROWS_PER_GROUP),
                             ROWS_PER_WORKER // ROWS_PER_GROUP)],
        )

    return gmf(ug, ig, uro, iro, uemb128, iemb128).reshape(BATCH, EMB)


# speedup vs baseline: 5.1155x; 5.1155x over previous
import functools

import jax
import jax.numpy as jnp
from jax import lax
from jax.experimental import pallas as pl
from jax.experimental.pallas import tpu as pltpu
from jax.experimental.pallas import tpu_sc as plsc

BATCH = 16384
EMB = 16
LANES = 128
NUM_CORES = 2
NUM_SUBCORES = 16
NUM_WORKERS = NUM_CORES * NUM_SUBCORES
ROWS_PER_WORKER = BATCH // NUM_WORKERS
GROUP = 8


def kernel(user_idx, item_idx, user_emb, item_emb):
    uembT = user_emb.T
    iembT = item_emb.T

    mesh = plsc.VectorSubcoreMesh(core_axis_name="c", subcore_axis_name="s")

    @functools.partial(
        pl.kernel,
        out_type=jax.ShapeDtypeStruct((EMB, BATCH), jnp.float32),
        mesh=mesh,
        compiler_params=pltpu.CompilerParams(needs_layout_passes=False),
        scratch_types=[
            pltpu.VMEM((ROWS_PER_WORKER,), jnp.int32),
            pltpu.VMEM((ROWS_PER_WORKER,), jnp.int32),
            pltpu.VMEM((2, GROUP, EMB, LANES), jnp.float32),
            pltpu.VMEM((2, GROUP, EMB, LANES), jnp.float32),
            pltpu.VMEM((EMB, ROWS_PER_WORKER), jnp.float32),
            pltpu.SemaphoreType.DMA,
            pltpu.SemaphoreType.DMA,
        ],
    )
    def gmf(uix_hbm, iix_hbm, uembT_hbm, iembT_hbm, outT_hbm,
            uix_v, iix_v, ublk, iblk, out_v, sem_u, sem_i):
        wid = lax.axis_index("s") * NUM_CORES + lax.axis_index("c")
        base = wid * ROWS_PER_WORKER
        pltpu.sync_copy(uix_hbm.at[pl.ds(base, ROWS_PER_WORKER)], uix_v)
        pltpu.sync_copy(iix_hbm.at[pl.ds(base, ROWS_PER_WORKER)], iix_v)

        iota16 = lax.broadcasted_iota(jnp.int32, (16,), 0)

        @pl.loop(0, ROWS_PER_WORKER, step=2 * GROUP)
        def _(r0):
            uix16 = uix_v[pl.ds(r0, 16)]
            iix16 = iix_v[pl.ds(r0, 16)]
            for half in range(2):
                waits = []
                for j in range(GROUP):
                    jj = half * GROUP + j
                    c0u = pl.multiple_of(uix16[jj] & -LANES, LANES)
                    c0i = pl.multiple_of(iix16[jj] & -LANES, LANES)
                    cu = pltpu.async_copy(
                        uembT_hbm.at[:, pl.ds(c0u, LANES)],
                        ublk.at[half].at[j], sem_u)
                    ci = pltpu.async_copy(
                        iembT_hbm.at[:, pl.ds(c0i, LANES)],
                        iblk.at[half].at[j], sem_i)
                    waits.append((cu, ci))
                for cu, ci in waits:
                    cu.wait()
                    ci.wait()
                for j in range(GROUP):
                    jj = half * GROUP + j
                    ucol = jnp.broadcast_to(uix16[jj] & (LANES - 1), (16,))
                    icol = jnp.broadcast_to(iix16[jj] & (LANES - 1), (16,))
                    u16 = plsc.load_gather(ublk.at[half].at[j], [iota16, ucol])
                    i16 = plsc.load_gather(iblk.at[half].at[j], [iota16, icol])
                    ocol = jnp.broadcast_to(r0 + jj, (16,))
                    plsc.store_scatter(out_v, [iota16, ocol], u16 * i16)

        pltpu.sync_copy(out_v, outT_hbm.at[:, pl.ds(base, ROWS_PER_WORKER)])

    return gmf(user_idx, item_idx, uembT, iembT).T


# speedup vs baseline: 6.6470x; 1.2994x over previous
import functools

import jax
import jax.numpy as jnp
from jax import lax
from jax.experimental import pallas as pl
from jax.experimental.pallas import tpu as pltpu
from jax.experimental.pallas import tpu_sc as plsc

BATCH = 16384
EMB = 16
LANES = 128
NUM_CORES = 2
NUM_SUBCORES = 16
NUM_WORKERS = NUM_CORES * NUM_SUBCORES
ROWS_PER_WORKER = BATCH // NUM_WORKERS
GROUP = 8


def kernel(user_idx, item_idx, user_emb, item_emb):
    uembT = user_emb.T
    iembT = item_emb.T

    mesh = plsc.VectorSubcoreMesh(core_axis_name="c", subcore_axis_name="s")

    @functools.partial(
        pl.kernel,
        out_type=jax.ShapeDtypeStruct((EMB, BATCH), jnp.float32),
        mesh=mesh,
        compiler_params=pltpu.CompilerParams(needs_layout_passes=False),
        scratch_types=[
            pltpu.VMEM((ROWS_PER_WORKER,), jnp.int32),
            pltpu.VMEM((ROWS_PER_WORKER,), jnp.int32),
            pltpu.VMEM((2, GROUP, EMB, LANES), jnp.float32),
            pltpu.VMEM((2, GROUP, EMB, LANES), jnp.float32),
            pltpu.VMEM((EMB, ROWS_PER_WORKER), jnp.float32),
            pltpu.SemaphoreType.DMA,
            pltpu.SemaphoreType.DMA,
        ],
    )
    def gmf(uix_hbm, iix_hbm, uembT_hbm, iembT_hbm, outT_hbm,
            uix_v, iix_v, ublk, iblk, out_v, sem_u, sem_i):
        wid = lax.axis_index("s") * NUM_CORES + lax.axis_index("c")
        base = wid * ROWS_PER_WORKER
        pltpu.sync_copy(uix_hbm.at[pl.ds(base, ROWS_PER_WORKER)], uix_v)
        pltpu.sync_copy(iix_hbm.at[pl.ds(base, ROWS_PER_WORKER)], iix_v)

        iota16 = lax.broadcasted_iota(jnp.int32, (16,), 0)

        def window_copies(uix16, iix16, half, buf):
            copies = []
            for j in range(GROUP):
                jj = half * GROUP + j
                c0u = pl.multiple_of(uix16[jj] & -LANES, LANES)
                c0i = pl.multiple_of(iix16[jj] & -LANES, LANES)
                copies.append((
                    pltpu.make_async_copy(
                        uembT_hbm.at[:, pl.ds(c0u, LANES)],
                        ublk.at[buf].at[j], sem_u),
                    pltpu.make_async_copy(
                        iembT_hbm.at[:, pl.ds(c0i, LANES)],
                        iblk.at[buf].at[j], sem_i),
                ))
            return copies

        def fire(uix16, iix16, half, buf):
            for cu, ci in window_copies(uix16, iix16, half, buf):
                cu.start()
                ci.start()

        def drain(uix16, iix16, half, buf):
            for cu, ci in window_copies(uix16, iix16, half, buf):
                cu.wait()
                ci.wait()

        def compute(uix16, iix16, r0, half, buf):
            for j in range(GROUP):
                jj = half * GROUP + j
                ucol = jnp.broadcast_to(uix16[jj] & (LANES - 1), (16,))
                icol = jnp.broadcast_to(iix16[jj] & (LANES - 1), (16,))
                u16 = plsc.load_gather(ublk.at[buf].at[j], [iota16, ucol])
                i16 = plsc.load_gather(iblk.at[buf].at[j], [iota16, icol])
                ocol = jnp.broadcast_to(r0 + jj, (16,))
                plsc.store_scatter(out_v, [iota16, ocol], u16 * i16)

        uix16p = uix_v[pl.ds(0, 16)]
        iix16p = iix_v[pl.ds(0, 16)]
        fire(uix16p, iix16p, 0, 0)

        @pl.loop(0, ROWS_PER_WORKER, step=2 * GROUP)
        def _(r0):
            uix16 = uix_v[pl.ds(r0, 16)]
            iix16 = iix_v[pl.ds(r0, 16)]
            fire(uix16, iix16, 1, 1)
            drain(uix16, iix16, 0, 0)
            compute(uix16, iix16, r0, 0, 0)

            @pl.when(r0 + 2 * GROUP < ROWS_PER_WORKER)
            def _():
                uix16n = uix_v[pl.ds(r0 + 16, 16)]
                iix16n = iix_v[pl.ds(r0 + 16, 16)]
                fire(uix16n, iix16n, 0, 0)

            drain(uix16, iix16, 1, 1)
            compute(uix16, iix16, r0, 1, 1)

        pltpu.sync_copy(out_v, outT_hbm.at[:, pl.ds(base, ROWS_PER_WORKER)])

    return gmf(user_idx, item_idx, uembT, iembT).T


# speedup vs baseline: 6.6851x; 1.0057x over previous
import functools

import jax
import jax.numpy as jnp
from jax import lax
from jax.experimental import pallas as pl
from jax.experimental.pallas import tpu as pltpu
from jax.experimental.pallas import tpu_sc as plsc

BATCH = 16384
EMB = 16
LANES = 128
NUM_CORES = 2
NUM_SUBCORES = 16
NUM_WORKERS = NUM_CORES * NUM_SUBCORES
ROWS_PER_WORKER = BATCH // NUM_WORKERS
GROUP = 8


def kernel(user_idx, item_idx, user_emb, item_emb):
    uembT = user_emb.T
    iembT = item_emb.T

    mesh = plsc.VectorSubcoreMesh(core_axis_name="c", subcore_axis_name="s")

    @functools.partial(
        pl.kernel,
        out_type=jax.ShapeDtypeStruct((EMB, BATCH), jnp.float32),
        mesh=mesh,
        compiler_params=pltpu.CompilerParams(needs_layout_passes=False),
        scratch_types=[
            pltpu.VMEM((ROWS_PER_WORKER,), jnp.int32),
            pltpu.VMEM((ROWS_PER_WORKER,), jnp.int32),
            pltpu.VMEM((2, GROUP, EMB, LANES), jnp.float32),
            pltpu.VMEM((2, GROUP, EMB, LANES), jnp.float32),
            pltpu.VMEM((EMB, ROWS_PER_WORKER), jnp.float32),
            pltpu.SemaphoreType.DMA,
            pltpu.SemaphoreType.DMA,
        ],
    )
    def gmf(uix_hbm, iix_hbm, uembT_hbm, iembT_hbm, outT_hbm,
            uix_v, iix_v, ublk, iblk, out_v, sem_u, sem_i):
        wid = lax.axis_index("s") * NUM_CORES + lax.axis_index("c")
        base = wid * ROWS_PER_WORKER
        pltpu.sync_copy(uix_hbm.at[pl.ds(base, ROWS_PER_WORKER)], uix_v)
        pltpu.sync_copy(iix_hbm.at[pl.ds(base, ROWS_PER_WORKER)], iix_v)

        iota16 = lax.broadcasted_iota(jnp.int32, (16,), 0)

        def fire(uc016, ic016, half, buf):
            for j in range(GROUP):
                jj = half * GROUP + j
                c0u = pl.multiple_of(uc016[jj], LANES)
                c0i = pl.multiple_of(ic016[jj], LANES)
                pltpu.make_async_copy(
                    uembT_hbm.at[:, pl.ds(c0u, LANES)],
                    ublk.at[buf].at[j], sem_u).start()
                pltpu.make_async_copy(
                    iembT_hbm.at[:, pl.ds(c0i, LANES)],
                    iblk.at[buf].at[j], sem_i).start()

        def drain(buf):
            dummy = uembT_hbm.at[:, pl.ds(0, LANES)]
            for j in range(GROUP):
                pltpu.make_async_copy(dummy, ublk.at[buf].at[j], sem_u).wait()
                pltpu.make_async_copy(dummy, iblk.at[buf].at[j], sem_i).wait()

        def compute(ul16, il16, r0, half, buf):
            for j in range(GROUP):
                jj = half * GROUP + j
                ucol = jnp.broadcast_to(ul16[jj], (16,))
                icol = jnp.broadcast_to(il16[jj], (16,))
                u16 = plsc.load_gather(ublk.at[buf].at[j], [iota16, ucol])
                i16 = plsc.load_gather(iblk.at[buf].at[j], [iota16, icol])
                ocol = jnp.broadcast_to(r0 + jj, (16,))
                plsc.store_scatter(out_v, [iota16, ocol], u16 * i16)

        def split(uix16, iix16):
            return (uix16 & -LANES, iix16 & -LANES,
                    uix16 & (LANES - 1), iix16 & (LANES - 1))

        uc0p, ic0p, _, _ = split(uix_v[pl.ds(0, 16)], iix_v[pl.ds(0, 16)])
        fire(uc0p, ic0p, 0, 0)

        @pl.loop(0, ROWS_PER_WORKER, step=2 * GROUP)
        def _(r0):
            uc016, ic016, ul16, il16 = split(
                uix_v[pl.ds(r0, 16)], iix_v[pl.ds(r0, 16)])
            fire(uc016, ic016, 1, 1)
            drain(0)
            compute(ul16, il16, r0, 0, 0)

            @pl.when(r0 + 2 * GROUP < ROWS_PER_WORKER)
            def _():
                uc016n, ic016n, _, _ = split(
                    uix_v[pl.ds(r0 + 16, 16)], iix_v[pl.ds(r0 + 16, 16)])
                fire(uc016n, ic016n, 0, 0)

            drain(1)
            compute(ul16, il16, r0, 1, 1)

        pltpu.sync_copy(out_v, outT_hbm.at[:, pl.ds(base, ROWS_PER_WORKER)])

    return gmf(user_idx, item_idx, uembT, iembT).T


# speedup vs baseline: 6.8355x; 1.0225x over previous
import functools

import jax
import jax.numpy as jnp
from jax import lax
from jax.experimental import pallas as pl
from jax.experimental.pallas import tpu as pltpu
from jax.experimental.pallas import tpu_sc as plsc

BATCH = 16384
EMB = 16
LANES = 128
NUM_CORES = 2
NUM_SUBCORES = 16
NUM_WORKERS = NUM_CORES * NUM_SUBCORES
ROWS_PER_WORKER = BATCH // NUM_WORKERS
GROUP = 8


def kernel(user_idx, item_idx, user_emb, item_emb):
    uembT = user_emb.T
    iembT = item_emb.T

    mesh = plsc.VectorSubcoreMesh(core_axis_name="c", subcore_axis_name="s")

    @functools.partial(
        pl.kernel,
        out_type=jax.ShapeDtypeStruct((EMB, BATCH), jnp.float32),
        mesh=mesh,
        compiler_params=pltpu.CompilerParams(needs_layout_passes=False),
        scratch_types=[
            pltpu.VMEM((ROWS_PER_WORKER,), jnp.int32),
            pltpu.VMEM((ROWS_PER_WORKER,), jnp.int32),
            pltpu.VMEM((2, GROUP * EMB, LANES), jnp.float32),
            pltpu.VMEM((2, GROUP * EMB, LANES), jnp.float32),
            pltpu.VMEM((EMB, ROWS_PER_WORKER), jnp.float32),
            pltpu.SemaphoreType.DMA,
            pltpu.SemaphoreType.DMA,
        ],
    )
    def gmf(uix_hbm, iix_hbm, uembT_hbm, iembT_hbm, outT_hbm,
            uix_v, iix_v, ublk, iblk, out_v, sem_u, sem_i):
        wid = lax.axis_index("s") * NUM_CORES + lax.axis_index("c")
        base = wid * ROWS_PER_WORKER
        pltpu.sync_copy(uix_hbm.at[pl.ds(base, ROWS_PER_WORKER)], uix_v)
        pltpu.sync_copy(iix_hbm.at[pl.ds(base, ROWS_PER_WORKER)], iix_v)

        iota16 = lax.broadcasted_iota(jnp.int32, (16,), 0)
        slot8 = iota16 & 7
        dhalf = iota16 >> 3

        def fire(uc016, ic016, half, buf):
            for j in range(GROUP):
                jj = half * GROUP + j
                c0u = pl.multiple_of(uc016[jj], LANES)
                c0i = pl.multiple_of(ic016[jj], LANES)
                pltpu.make_async_copy(
                    uembT_hbm.at[:, pl.ds(c0u, LANES)],
                    ublk.at[buf].at[pl.ds(j * EMB, EMB)], sem_u).start()
                pltpu.make_async_copy(
                    iembT_hbm.at[:, pl.ds(c0i, LANES)],
                    iblk.at[buf].at[pl.ds(j * EMB, EMB)], sem_i).start()

        def drain(buf):
            dummy = uembT_hbm.at[:, pl.ds(0, LANES)]
            for j in range(GROUP):
                pltpu.make_async_copy(
                    dummy, ublk.at[buf].at[pl.ds(j * EMB, EMB)], sem_u).wait()
                pltpu.make_async_copy(
                    dummy, iblk.at[buf].at[pl.ds(j * EMB, EMB)], sem_i).wait()

        def compute(r0, half, buf):
            rbase = r0 + half * GROUP
            rows8 = jnp.broadcast_to(rbase, (16,)) + slot8
            ulanes = plsc.load_gather(uix_v, [rows8]) & (LANES - 1)
            ilanes = plsc.load_gather(iix_v, [rows8]) & (LANES - 1)
            for d0 in range(0, EMB, 2):
                brow = slot8 * EMB + dhalf + d0
                u16 = plsc.load_gather(ublk.at[buf], [brow, ulanes])
                i16 = plsc.load_gather(iblk.at[buf], [brow, ilanes])
                plsc.store_scatter(out_v, [dhalf + d0, rows8], u16 * i16)

        def split_c0(uix16, iix16):
            return uix16 & -LANES, iix16 & -LANES

        uc0p, ic0p = split_c0(uix_v[pl.ds(0, 16)], iix_v[pl.ds(0, 16)])
        fire(uc0p, ic0p, 0, 0)

        @pl.loop(0, ROWS_PER_WORKER, step=2 * GROUP)
        def _(r0):
            uc016, ic016 = split_c0(uix_v[pl.ds(r0, 16)], iix_v[pl.ds(r0, 16)])
            fire(uc016, ic016, 1, 1)
            drain(0)
            compute(r0, 0, 0)

            @pl.when(r0 + 2 * GROUP < ROWS_PER_WORKER)
            def _():
                uc016n, ic016n = split_c0(
                    uix_v[pl.ds(r0 + 16, 16)], iix_v[pl.ds(r0 + 16, 16)])
                fire(uc016n, ic016n, 0, 0)

            drain(1)
            compute(r0, 1, 1)

        pltpu.sync_copy(out_v, outT_hbm.at[:, pl.ds(base, ROWS_PER_WORKER)])

    return gmf(user_idx, item_idx, uembT, iembT).T


# speedup vs baseline: 7.2141x; 1.0554x over previous
import functools

import jax
import jax.numpy as jnp
from jax import lax
from jax.experimental import pallas as pl
from jax.experimental.pallas import tpu as pltpu
from jax.experimental.pallas import tpu_sc as plsc

BATCH = 16384
EMB = 16
LANES = 128
NUM_CORES = 2
NUM_SUBCORES = 16
NUM_WORKERS = NUM_CORES * NUM_SUBCORES
ROWS_PER_WORKER = BATCH // NUM_WORKERS
GROUP = 8


def kernel(user_idx, item_idx, user_emb, item_emb):
    uembT = user_emb.T
    iembT = item_emb.T

    mesh = plsc.VectorSubcoreMesh(core_axis_name="c", subcore_axis_name="s")

    @functools.partial(
        pl.kernel,
        out_type=jax.ShapeDtypeStruct((EMB, BATCH), jnp.float32),
        mesh=mesh,
        compiler_params=pltpu.CompilerParams(needs_layout_passes=False),
        scratch_types=[
            pltpu.VMEM((ROWS_PER_WORKER + 16,), jnp.int32),
            pltpu.VMEM((ROWS_PER_WORKER + 16,), jnp.int32),
            pltpu.VMEM((3, GROUP * EMB, LANES), jnp.float32),
            pltpu.VMEM((3, GROUP * EMB, LANES), jnp.float32),
            pltpu.VMEM((EMB, ROWS_PER_WORKER), jnp.float32),
            pltpu.SemaphoreType.DMA,
            pltpu.SemaphoreType.DMA,
        ],
    )
    def gmf(uix_hbm, iix_hbm, uembT_hbm, iembT_hbm, outT_hbm,
            uix_v, iix_v, ublk, iblk, out_v, sem_u, sem_i):
        wid = lax.axis_index("s") * NUM_CORES + lax.axis_index("c")
        base = wid * ROWS_PER_WORKER
        pltpu.sync_copy(uix_hbm.at[pl.ds(base, ROWS_PER_WORKER)],
                        uix_v.at[pl.ds(0, ROWS_PER_WORKER)])
        pltpu.sync_copy(iix_hbm.at[pl.ds(base, ROWS_PER_WORKER)],
                        iix_v.at[pl.ds(0, ROWS_PER_WORKER)])

        iota16 = lax.broadcasted_iota(jnp.int32, (16,), 0)
        slot8 = iota16 & 7
        dhalf = iota16 >> 3

        def fire(g, buf):
            uc016 = uix_v[pl.ds(g * GROUP, 16)] & -LANES
            ic016 = iix_v[pl.ds(g * GROUP, 16)] & -LANES
            for j in range(GROUP):
                c0u = pl.multiple_of(uc016[j], LANES)
                c0i = pl.multiple_of(ic016[j], LANES)
                pltpu.make_async_copy(
                    uembT_hbm.at[:, pl.ds(c0u, LANES)],
                    ublk.at[buf].at[pl.ds(j * EMB, EMB)], sem_u).start()
                pltpu.make_async_copy(
                    iembT_hbm.at[:, pl.ds(c0i, LANES)],
                    iblk.at[buf].at[pl.ds(j * EMB, EMB)], sem_i).start()

        def drain(buf):
            dummy = uembT_hbm.at[:, pl.ds(0, LANES)]
            for j in range(GROUP):
                pltpu.make_async_copy(
                    dummy, ublk.at[buf].at[pl.ds(j * EMB, EMB)], sem_u).wait()
                pltpu.make_async_copy(
                    dummy, iblk.at[buf].at[pl.ds(j * EMB, EMB)], sem_i).wait()

        def compute(g, buf):
            rows8 = jnp.broadcast_to(g * GROUP, (16,)) + slot8
            ulanes = plsc.load_gather(uix_v, [rows8]) & (LANES - 1)
            ilanes = plsc.load_gather(iix_v, [rows8]) & (LANES - 1)
            for d0 in range(0, EMB, 2):
                brow = slot8 * EMB + dhalf + d0
                u16 = plsc.load_gather(ublk.at[buf], [brow, ulanes])
                i16 = plsc.load_gather(iblk.at[buf], [brow, ilanes])
                plsc.store_scatter(out_v, [dhalf + d0, rows8], u16 * i16)

        NUM_GROUPS = ROWS_PER_WORKER // GROUP

        def fire_if_valid(g, buf):
            @pl.when(g < NUM_GROUPS)
            def _():
                fire(g, buf)

        fire(0, 0)
        fire(1, 1)

        @pl.loop(0, (NUM_GROUPS - 1) // 3)
        def _(k):
            g0 = k * 3
            fire(g0 + 2, 2)
            drain(0)
            compute(g0, 0)
            fire_if_valid(g0 + 3, 0)
            drain(1)
            compute(g0 + 1, 1)
            fire_if_valid(g0 + 4, 1)
            drain(2)
            compute(g0 + 2, 2)

        drain(0)
        compute(NUM_GROUPS - 1, 0)

        pltpu.sync_copy(out_v, outT_hbm.at[:, pl.ds(base, ROWS_PER_WORKER)])

    return gmf(user_idx, item_idx, uembT, iembT).T
